# baseline (speedup 1.0000x reference)
import functools

import jax
import jax.numpy as jnp
from jax import lax
from jax.experimental import pallas as pl
from jax.experimental.pallas import tpu as pltpu
from jax.experimental.pallas import tpu_sc as plsc

_NC = 2
_NS = 16
_NW = _NC * _NS

_B = 16384
_D = 64
_H = 128
_BPW = _B // _NW
_CHUNK = 128
_NCHUNK = _BPW // _CHUNK


def _pack_body(x_ref, o_ref):
  t = jnp.transpose(x_ref[...])
  o_ref[...] = jnp.concatenate([t[:512], t[512:]], axis=1)


def _tc_pack(table_t):
  n = table_t.shape[1]
  grid = (n + 1023) // 1024
  return pl.pallas_call(
      _pack_body,
      grid=(grid,),
      in_specs=[pl.BlockSpec((_D, 1024), lambda i: (0, i))],
      out_specs=pl.BlockSpec((512, 2 * _D), lambda i: (i, 0)),
      out_shape=jax.ShapeDtypeStruct((grid * 512, 2 * _D), jnp.float32),
  )(table_t)


def _sc_gather(uidx2d, cidx2d, ut2, ct2):
  mesh = plsc.VectorSubcoreMesh(core_axis_name="c", subcore_axis_name="s")

  @functools.partial(
      pl.kernel,
      out_type=(
          jax.ShapeDtypeStruct((_B, 2 * _D), jnp.float32),
          jax.ShapeDtypeStruct((_B, 2 * _D), jnp.float32),
      ),
      mesh=mesh,
      scratch_types=[
          pltpu.VMEM((_NCHUNK, _CHUNK), jnp.int32),
          pltpu.VMEM((_NCHUNK, _CHUNK), jnp.int32),
          pltpu.VMEM((2 * _CHUNK, 2 * _D), jnp.float32),
          pltpu.VMEM((2 * _CHUNK, 2 * _D), jnp.float32),
          pltpu.SemaphoreType.DMA,
      ],
  )
  def k(uidx_hbm, cidx_hbm, ut_hbm, ct_hbm, u_out, c_out,
        uidx_v, cidx_v, uwin_v, cwin_v, sem):
    wid = lax.axis_index("s") * _NC + lax.axis_index("c")
    base = wid * _BPW
    pltpu.sync_copy(uidx_hbm.at[pl.ds(wid * _NCHUNK, _NCHUNK)], uidx_v)
    pltpu.sync_copy(cidx_hbm.at[pl.ds(wid * _NCHUNK, _NCHUNK)], cidx_v)
    for h in range(2):
      copies = []
      for ch in range(2):
        copies.append(pltpu.async_copy(
            ut_hbm.at[uidx_v.at[2 * h + ch]],
            uwin_v.at[pl.ds(ch * _CHUNK, _CHUNK)], sem))
        copies.append(pltpu.async_copy(
            ct_hbm.at[cidx_v.at[2 * h + ch]],
            cwin_v.at[pl.ds(ch * _CHUNK, _CHUNK)], sem))
      for cp in copies:
        cp.wait()
      pltpu.sync_copy(uwin_v, u_out.at[pl.ds(base + h * 2 * _CHUNK,
                                             2 * _CHUNK)])
      pltpu.sync_copy(cwin_v, c_out.at[pl.ds(base + h * 2 * _CHUNK,
                                             2 * _CHUNK)])

  return k(uidx2d, cidx2d, ut2, ct2)


def _mlp_body(xu_ref, xc_ref, u_ref, c_ref, w1_ref, b1_ref, w2_ref, b2_ref,
              o_ref):
  usel = jnp.where(u_ref[...] == 1, xu_ref[:, _D:], xu_ref[:, :_D])
  csel = jnp.where(c_ref[...] == 1, xc_ref[:, _D:], xc_ref[:, :_D])
  x = jnp.concatenate([usel, csel], axis=1)
  h = lax.dot_general(x, w1_ref[...], (((1,), (1,)), ((), ())),
                      preferred_element_type=jnp.float32)
  h = jnp.maximum(h + b1_ref[...], 0.0)
  o_ref[...] = jnp.sum(h * w2_ref[...], axis=1, keepdims=True) + b2_ref[...]


def _tc_mlp(xu, xc, u_ids, c_ids, w1, b1, w2, b2):
  rows = 2048
  grid = _B // rows
  return pl.pallas_call(
      _mlp_body,
      grid=(grid,),
      in_specs=[
          pl.BlockSpec((rows, 2 * _D), lambda i: (i, 0)),
          pl.BlockSpec((rows, 2 * _D), lambda i: (i, 0)),
          pl.BlockSpec((rows, 1), lambda i: (i, 0)),
          pl.BlockSpec((rows, 1), lambda i: (i, 0)),
          pl.BlockSpec((_H, 2 * _D), lambda i: (0, 0)),
          pl.BlockSpec((1, _H), lambda i: (0, 0)),
          pl.BlockSpec((1, _H), lambda i: (0, 0)),
          pl.BlockSpec((1, 1), lambda i: (0, 0)),
      ],
      out_specs=pl.BlockSpec((rows, 1), lambda i: (i, 0)),
      out_shape=jax.ShapeDtypeStruct((_B, 1), jnp.float32),
  )(xu, xc, u_ids, c_ids, w1, b1, w2, b2)


def kernel(users, courses, user_table, course_table, W1, b1, W2, b2):
  users = users.astype(jnp.int32)
  courses = courses.astype(jnp.int32)
  uidx2d = (((users >> 10) << 9) + (users & 511)).reshape(
      _NW * _NCHUNK, _CHUNK)
  cidx2d = (((courses >> 10) << 9) + (courses & 511)).reshape(
      _NW * _NCHUNK, _CHUNK)
  uhalf = ((users >> 9) & 1).reshape(_B, 1)
  chalf = ((courses >> 9) & 1).reshape(_B, 1)
  ut2 = _tc_pack(user_table.T)
  ct2 = _tc_pack(course_table.T)
  xu, xc = _sc_gather(uidx2d, cidx2d, ut2, ct2)
  return _tc_mlp(xu, xc, uhalf, chalf,
                 W1, b1.reshape(1, _H), W2, b2.reshape(1, 1))


# speedup vs baseline: 1.9629x; 1.9629x over previous
import functools

import jax
import jax.numpy as jnp
from jax import lax
from jax.experimental import pallas as pl
from jax.experimental.pallas import tpu as pltpu
from jax.experimental.pallas import tpu_sc as plsc

_NC = 2
_NS = 16
_NW = _NC * _NS

_B = 16384
_D = 64
_H = 128
_BPW = _B // _NW
_CHUNK = 128
_NCHUNK = _BPW // _CHUNK


_PBLK = 4096
_PHALF = _PBLK // 2


def _pack_body(x_ref, o_ref):
  t = jnp.transpose(x_ref[...])
  o_ref[...] = jnp.concatenate([t[:_PHALF], t[_PHALF:]], axis=1)


def _tc_pack(table_t):
  n = table_t.shape[1]
  grid = (n + _PBLK - 1) // _PBLK
  return pl.pallas_call(
      _pack_body,
      grid=(grid,),
      in_specs=[pl.BlockSpec((_D, _PBLK), lambda i: (0, i))],
      out_specs=pl.BlockSpec((_PHALF, 2 * _D), lambda i: (i, 0)),
      out_shape=jax.ShapeDtypeStruct((grid * _PHALF, 2 * _D), jnp.float32),
  )(table_t)


def _sc_gather(uidx2d, cidx2d, ut2, ct2):
  mesh = plsc.VectorSubcoreMesh(core_axis_name="c", subcore_axis_name="s")

  @functools.partial(
      pl.kernel,
      out_type=(
          jax.ShapeDtypeStruct((_B, 2 * _D), jnp.float32),
          jax.ShapeDtypeStruct((_B, 2 * _D), jnp.float32),
      ),
      mesh=mesh,
      scratch_types=[
          pltpu.VMEM((_NCHUNK, _CHUNK), jnp.int32),
          pltpu.VMEM((_NCHUNK, _CHUNK), jnp.int32),
          pltpu.VMEM((2 * _CHUNK, 2 * _D), jnp.float32),
          pltpu.VMEM((2 * _CHUNK, 2 * _D), jnp.float32),
          pltpu.SemaphoreType.DMA,
      ],
  )
  def k(uidx_hbm, cidx_hbm, ut_hbm, ct_hbm, u_out, c_out,
        uidx_v, cidx_v, uwin_v, cwin_v, sem):
    wid = lax.axis_index("s") * _NC + lax.axis_index("c")
    base = wid * _BPW
    pltpu.sync_copy(uidx_hbm.at[pl.ds(wid * _NCHUNK, _NCHUNK)], uidx_v)
    pltpu.sync_copy(cidx_hbm.at[pl.ds(wid * _NCHUNK, _NCHUNK)], cidx_v)
    for h in range(2):
      copies = []
      for ch in range(2):
        copies.append(pltpu.async_copy(
            ut_hbm.at[uidx_v.at[2 * h + ch]],
            uwin_v.at[pl.ds(ch * _CHUNK, _CHUNK)], sem))
        copies.append(pltpu.async_copy(
            ct_hbm.at[cidx_v.at[2 * h + ch]],
            cwin_v.at[pl.ds(ch * _CHUNK, _CHUNK)], sem))
      for cp in copies:
        cp.wait()
      pltpu.sync_copy(uwin_v, u_out.at[pl.ds(base + h * 2 * _CHUNK,
                                             2 * _CHUNK)])
      pltpu.sync_copy(cwin_v, c_out.at[pl.ds(base + h * 2 * _CHUNK,
                                             2 * _CHUNK)])

  return k(uidx2d, cidx2d, ut2, ct2)


def _mlp_body(xu_ref, xc_ref, u_ref, c_ref, w1_ref, b1_ref, w2_ref, b2_ref,
              o_ref):
  usel = jnp.where(u_ref[...] == 1, xu_ref[:, _D:], xu_ref[:, :_D])
  csel = jnp.where(c_ref[...] == 1, xc_ref[:, _D:], xc_ref[:, :_D])
  x = jnp.concatenate([usel, csel], axis=1)
  h = lax.dot_general(x, w1_ref[...], (((1,), (1,)), ((), ())),
                      preferred_element_type=jnp.float32)
  h = jnp.maximum(h + b1_ref[...], 0.0)
  o_ref[...] = jnp.sum(h * w2_ref[...], axis=1, keepdims=True) + b2_ref[...]


def _tc_mlp(xu, xc, u_ids, c_ids, w1, b1, w2, b2):
  rows = 2048
  grid = _B // rows
  return pl.pallas_call(
      _mlp_body,
      grid=(grid,),
      in_specs=[
          pl.BlockSpec((rows, 2 * _D), lambda i: (i, 0)),
          pl.BlockSpec((rows, 2 * _D), lambda i: (i, 0)),
          pl.BlockSpec((rows, 1), lambda i: (i, 0)),
          pl.BlockSpec((rows, 1), lambda i: (i, 0)),
          pl.BlockSpec((_H, 2 * _D), lambda i: (0, 0)),
          pl.BlockSpec((1, _H), lambda i: (0, 0)),
          pl.BlockSpec((1, _H), lambda i: (0, 0)),
          pl.BlockSpec((1, 1), lambda i: (0, 0)),
      ],
      out_specs=pl.BlockSpec((rows, 1), lambda i: (i, 0)),
      out_shape=jax.ShapeDtypeStruct((_B, 1), jnp.float32),
  )(xu, xc, u_ids, c_ids, w1, b1, w2, b2)


def kernel(users, courses, user_table, course_table, W1, b1, W2, b2):
  users = users.astype(jnp.int32)
  courses = courses.astype(jnp.int32)
  uidx2d = ((users // _PBLK) * _PHALF + (users % _PHALF)).reshape(
      _NW * _NCHUNK, _CHUNK)
  cidx2d = ((courses // _PBLK) * _PHALF + (courses % _PHALF)).reshape(
      _NW * _NCHUNK, _CHUNK)
  uhalf = ((users // _PHALF) & 1).reshape(_B, 1)
  chalf = ((courses // _PHALF) & 1).reshape(_B, 1)
  ut2 = _tc_pack(user_table.T)
  ct2 = _tc_pack(course_table.T)
  xu, xc = _sc_gather(uidx2d, cidx2d, ut2, ct2)
  return _tc_mlp(xu, xc, uhalf, chalf,
                 W1, b1.reshape(1, _H), W2, b2.reshape(1, 1))


# speedup vs baseline: 2.6377x; 1.3438x over previous
import functools

import jax
import jax.numpy as jnp
from jax import lax
from jax.experimental import pallas as pl
from jax.experimental.pallas import tpu as pltpu
from jax.experimental.pallas import tpu_sc as plsc

_NC = 2
_NS = 16
_NW = _NC * _NS

_B = 16384
_D = 64
_H = 128
_BPW = _B // _NW
_CHUNK = 128
_NCHUNK = _BPW // _CHUNK


_PBLK = 16384
_PHALF = _PBLK // 2


def _pack_body(x_ref, o_ref):
  t = jnp.transpose(x_ref[...])
  o_ref[...] = jnp.concatenate([t[:_PHALF], t[_PHALF:]], axis=1)


def _tc_pack(table_t):
  n = table_t.shape[1]
  grid = (n + _PBLK - 1) // _PBLK
  return pl.pallas_call(
      _pack_body,
      grid=(grid,),
      in_specs=[pl.BlockSpec((_D, _PBLK), lambda i: (0, i))],
      out_specs=pl.BlockSpec((_PHALF, 2 * _D), lambda i: (i, 0)),
      out_shape=jax.ShapeDtypeStruct((grid * _PHALF, 2 * _D), jnp.float32),
  )(table_t)


def _sc_gather(uidx2d, cidx2d, ut2, ct2):
  mesh = plsc.VectorSubcoreMesh(core_axis_name="c", subcore_axis_name="s")

  @functools.partial(
      pl.kernel,
      out_type=(
          jax.ShapeDtypeStruct((_B, 2 * _D), jnp.float32),
          jax.ShapeDtypeStruct((_B, 2 * _D), jnp.float32),
      ),
      mesh=mesh,
      scratch_types=[
          pltpu.VMEM((_NCHUNK, _CHUNK), jnp.int32),
          pltpu.VMEM((_NCHUNK, _CHUNK), jnp.int32),
          pltpu.VMEM((2 * _CHUNK, 2 * _D), jnp.float32),
          pltpu.VMEM((2 * _CHUNK, 2 * _D), jnp.float32),
          pltpu.SemaphoreType.DMA,
      ],
  )
  def k(uidx_hbm, cidx_hbm, ut_hbm, ct_hbm, u_out, c_out,
        uidx_v, cidx_v, uwin_v, cwin_v, sem):
    wid = lax.axis_index("s") * _NC + lax.axis_index("c")
    base = wid * _BPW
    pltpu.sync_copy(uidx_hbm.at[pl.ds(wid * _NCHUNK, _NCHUNK)], uidx_v)
    pltpu.sync_copy(cidx_hbm.at[pl.ds(wid * _NCHUNK, _NCHUNK)], cidx_v)
    for h in range(2):
      copies = []
      for ch in range(2):
        copies.append(pltpu.async_copy(
            ut_hbm.at[uidx_v.at[2 * h + ch]],
            uwin_v.at[pl.ds(ch * _CHUNK, _CHUNK)], sem))
        copies.append(pltpu.async_copy(
            ct_hbm.at[cidx_v.at[2 * h + ch]],
            cwin_v.at[pl.ds(ch * _CHUNK, _CHUNK)], sem))
      for cp in copies:
        cp.wait()
      pltpu.sync_copy(uwin_v, u_out.at[pl.ds(base + h * 2 * _CHUNK,
                                             2 * _CHUNK)])
      pltpu.sync_copy(cwin_v, c_out.at[pl.ds(base + h * 2 * _CHUNK,
                                             2 * _CHUNK)])

  return k(uidx2d, cidx2d, ut2, ct2)


def _mlp_body(xu_ref, xc_ref, u_ref, c_ref, w1_ref, b1_ref, w2_ref, b2_ref,
              o_ref):
  usel = jnp.where(u_ref[...] == 1, xu_ref[:, _D:], xu_ref[:, :_D])
  csel = jnp.where(c_ref[...] == 1, xc_ref[:, _D:], xc_ref[:, :_D])
  x = jnp.concatenate([usel, csel], axis=1)
  h = lax.dot_general(x, w1_ref[...], (((1,), (1,)), ((), ())),
                      preferred_element_type=jnp.float32)
  h = jnp.maximum(h + b1_ref[...], 0.0)
  o_ref[...] = jnp.sum(h * w2_ref[...], axis=1, keepdims=True) + b2_ref[...]


def _tc_mlp(xu, xc, u_ids, c_ids, w1, b1, w2, b2):
  rows = 2048
  grid = _B // rows
  return pl.pallas_call(
      _mlp_body,
      grid=(grid,),
      in_specs=[
          pl.BlockSpec((rows, 2 * _D), lambda i: (i, 0)),
          pl.BlockSpec((rows, 2 * _D), lambda i: (i, 0)),
          pl.BlockSpec((rows, 1), lambda i: (i, 0)),
          pl.BlockSpec((rows, 1), lambda i: (i, 0)),
          pl.BlockSpec((_H, 2 * _D), lambda i: (0, 0)),
          pl.BlockSpec((1, _H), lambda i: (0, 0)),
          pl.BlockSpec((1, _H), lambda i: (0, 0)),
          pl.BlockSpec((1, 1), lambda i: (0, 0)),
      ],
      out_specs=pl.BlockSpec((rows, 1), lambda i: (i, 0)),
      out_shape=jax.ShapeDtypeStruct((_B, 1), jnp.float32),
  )(xu, xc, u_ids, c_ids, w1, b1, w2, b2)


def kernel(users, courses, user_table, course_table, W1, b1, W2, b2):
  users = users.astype(jnp.int32)
  courses = courses.astype(jnp.int32)
  uidx2d = ((users // _PBLK) * _PHALF + (users % _PHALF)).reshape(
      _NW * _NCHUNK, _CHUNK)
  cidx2d = ((courses // _PBLK) * _PHALF + (courses % _PHALF)).reshape(
      _NW * _NCHUNK, _CHUNK)
  uhalf = ((users // _PHALF) & 1).reshape(_B, 1)
  chalf = ((courses // _PHALF) & 1).reshape(_B, 1)
  ut2 = _tc_pack(user_table.T)
  ct2 = _tc_pack(course_table.T)
  xu, xc = _sc_gather(uidx2d, cidx2d, ut2, ct2)
  return _tc_mlp(xu, xc, uhalf, chalf,
                 W1, b1.reshape(1, _H), W2, b2.reshape(1, 1))


# speedup vs baseline: 2.7217x; 1.0318x over previous
import functools

import jax
import jax.numpy as jnp
from jax import lax
from jax.experimental import pallas as pl
from jax.experimental.pallas import tpu as pltpu
from jax.experimental.pallas import tpu_sc as plsc

_NC = 2
_NS = 16
_NW = _NC * _NS

_B = 16384
_D = 64
_H = 128
_BPW = _B // _NW
_CHUNK = 128
_NCHUNK = _BPW // _CHUNK


_PBLK = 32768
_PHALF = _PBLK // 2


def _pack_body(x_ref, o_ref):
  x = x_ref[...]
  t_lo = jnp.transpose(x[:, :_PHALF])
  eye = jnp.eye(_D, dtype=jnp.float32)
  t_hi = lax.dot_general(x[:, _PHALF:], eye, (((0,), (0,)), ((), ())),
                         preferred_element_type=jnp.float32)
  o_ref[...] = jnp.concatenate([t_lo, t_hi], axis=1)


def _tc_pack(table_t):
  n = table_t.shape[1]
  grid = (n + _PBLK - 1) // _PBLK
  return pl.pallas_call(
      _pack_body,
      grid=(grid,),
      in_specs=[pl.BlockSpec((_D, _PBLK), lambda i: (0, i))],
      out_specs=pl.BlockSpec((_PHALF, 2 * _D), lambda i: (i, 0)),
      out_shape=jax.ShapeDtypeStruct((grid * _PHALF, 2 * _D), jnp.float32),
  )(table_t)


def _sc_gather(uidx2d, cidx2d, ut2, ct2):
  mesh = plsc.VectorSubcoreMesh(core_axis_name="c", subcore_axis_name="s")

  @functools.partial(
      pl.kernel,
      out_type=(
          jax.ShapeDtypeStruct((_B, 2 * _D), jnp.float32),
          jax.ShapeDtypeStruct((_B, 2 * _D), jnp.float32),
      ),
      mesh=mesh,
      scratch_types=[
          pltpu.VMEM((_NCHUNK, _CHUNK), jnp.int32),
          pltpu.VMEM((_NCHUNK, _CHUNK), jnp.int32),
          pltpu.VMEM((2 * _CHUNK, 2 * _D), jnp.float32),
          pltpu.VMEM((2 * _CHUNK, 2 * _D), jnp.float32),
          pltpu.SemaphoreType.DMA,
      ],
  )
  def k(uidx_hbm, cidx_hbm, ut_hbm, ct_hbm, u_out, c_out,
        uidx_v, cidx_v, uwin_v, cwin_v, sem):
    wid = lax.axis_index("s") * _NC + lax.axis_index("c")
    base = wid * _BPW
    pltpu.sync_copy(uidx_hbm.at[pl.ds(wid * _NCHUNK, _NCHUNK)], uidx_v)
    pltpu.sync_copy(cidx_hbm.at[pl.ds(wid * _NCHUNK, _NCHUNK)], cidx_v)
    for h in range(2):
      copies = []
      for ch in range(2):
        copies.append(pltpu.async_copy(
            ut_hbm.at[uidx_v.at[2 * h + ch]],
            uwin_v.at[pl.ds(ch * _CHUNK, _CHUNK)], sem))
        copies.append(pltpu.async_copy(
            ct_hbm.at[cidx_v.at[2 * h + ch]],
            cwin_v.at[pl.ds(ch * _CHUNK, _CHUNK)], sem))
      for cp in copies:
        cp.wait()
      pltpu.sync_copy(uwin_v, u_out.at[pl.ds(base + h * 2 * _CHUNK,
                                             2 * _CHUNK)])
      pltpu.sync_copy(cwin_v, c_out.at[pl.ds(base + h * 2 * _CHUNK,
                                             2 * _CHUNK)])

  return k(uidx2d, cidx2d, ut2, ct2)


def _mlp_body(xu_ref, xc_ref, u_ref, c_ref, w1_ref, b1_ref, w2_ref, b2_ref,
              o_ref):
  usel = jnp.where(u_ref[...] == 1, xu_ref[:, _D:], xu_ref[:, :_D])
  csel = jnp.where(c_ref[...] == 1, xc_ref[:, _D:], xc_ref[:, :_D])
  x = jnp.concatenate([usel, csel], axis=1)
  h = lax.dot_general(x, w1_ref[...], (((1,), (1,)), ((), ())),
                      preferred_element_type=jnp.float32)
  h = jnp.maximum(h + b1_ref[...], 0.0)
  o_ref[...] = jnp.sum(h * w2_ref[...], axis=1, keepdims=True) + b2_ref[...]


def _tc_mlp(xu, xc, u_ids, c_ids, w1, b1, w2, b2):
  rows = 2048
  grid = _B // rows
  return pl.pallas_call(
      _mlp_body,
      grid=(grid,),
      in_specs=[
          pl.BlockSpec((rows, 2 * _D), lambda i: (i, 0)),
          pl.BlockSpec((rows, 2 * _D), lambda i: (i, 0)),
          pl.BlockSpec((rows, 1), lambda i: (i, 0)),
          pl.BlockSpec((rows, 1), lambda i: (i, 0)),
          pl.BlockSpec((_H, 2 * _D), lambda i: (0, 0)),
          pl.BlockSpec((1, _H), lambda i: (0, 0)),
          pl.BlockSpec((1, _H), lambda i: (0, 0)),
          pl.BlockSpec((1, 1), lambda i: (0, 0)),
      ],
      out_specs=pl.BlockSpec((rows, 1), lambda i: (i, 0)),
      out_shape=jax.ShapeDtypeStruct((_B, 1), jnp.float32),
  )(xu, xc, u_ids, c_ids, w1, b1, w2, b2)


def kernel(users, courses, user_table, course_table, W1, b1, W2, b2):
  users = users.astype(jnp.int32)
  courses = courses.astype(jnp.int32)
  uidx2d = ((users // _PBLK) * _PHALF + (users % _PHALF)).reshape(
      _NW * _NCHUNK, _CHUNK)
  cidx2d = ((courses // _PBLK) * _PHALF + (courses % _PHALF)).reshape(
      _NW * _NCHUNK, _CHUNK)
  uhalf = ((users // _PHALF) & 1).reshape(_B, 1)
  chalf = ((courses // _PHALF) & 1).reshape(_B, 1)
  ut2 = _tc_pack(user_table.T)
  ct2 = _tc_pack(course_table.T)
  xu, xc = _sc_gather(uidx2d, cidx2d, ut2, ct2)
  return _tc_mlp(xu, xc, uhalf, chalf,
                 W1, b1.reshape(1, _H), W2, b2.reshape(1, 1))


# speedup vs baseline: 2.7691x; 1.0174x over previous
import functools

import jax
import jax.numpy as jnp
from jax import lax
from jax.experimental import pallas as pl
from jax.experimental.pallas import tpu as pltpu
from jax.experimental.pallas import tpu_sc as plsc

_NC = 2
_NS = 16
_NW = _NC * _NS

_B = 16384
_D = 64
_H = 128
_BPW = _B // _NW
_CHUNK = 128
_NCHUNK = _BPW // _CHUNK


_PBLK = 32768
_PHALF = _PBLK // 2


def _pack_body(x_ref, o_ref):
  x = x_ref[...]
  t_lo = jnp.transpose(x[:, :_PHALF])
  eye = jnp.eye(_D, dtype=jnp.float32)
  t_hi = lax.dot_general(x[:, _PHALF:], eye, (((0,), (0,)), ((), ())),
                         preferred_element_type=jnp.float32)
  o_ref[...] = jnp.concatenate([t_lo, t_hi], axis=1)


def _tc_pack(table_t):
  n = table_t.shape[1]
  grid = (n + _PBLK - 1) // _PBLK
  return pl.pallas_call(
      _pack_body,
      grid=(grid,),
      in_specs=[pl.BlockSpec((_D, _PBLK), lambda i: (0, i))],
      out_specs=pl.BlockSpec((_PHALF, 2 * _D), lambda i: (i, 0)),
      out_shape=jax.ShapeDtypeStruct((grid * _PHALF, 2 * _D), jnp.float32),
  )(table_t)


def _sc_gather(idx2d, tbl):
  mesh = plsc.VectorSubcoreMesh(core_axis_name="c", subcore_axis_name="s")

  @functools.partial(
      pl.kernel,
      out_type=jax.ShapeDtypeStruct((_B, 2 * _D), jnp.float32),
      mesh=mesh,
      scratch_types=[
          pltpu.VMEM((_NCHUNK, _CHUNK), jnp.int32),
          pltpu.VMEM((2 * _CHUNK, 2 * _D), jnp.float32),
          pltpu.SemaphoreType.DMA,
      ],
  )
  def k(idx_hbm, tbl_hbm, out, idx_v, win_v, sem):
    wid = lax.axis_index("s") * _NC + lax.axis_index("c")
    base = wid * _BPW
    pltpu.sync_copy(idx_hbm.at[pl.ds(wid * _NCHUNK, _NCHUNK)], idx_v)
    for h in range(2):
      copies = []
      for ch in range(2):
        copies.append(pltpu.async_copy(
            tbl_hbm.at[idx_v.at[2 * h + ch]],
            win_v.at[pl.ds(ch * _CHUNK, _CHUNK)], sem))
      for cp in copies:
        cp.wait()
      pltpu.sync_copy(win_v, out.at[pl.ds(base + h * 2 * _CHUNK,
                                          2 * _CHUNK)])

  return k(idx2d, tbl)


def _mlp_body(xu_ref, xc_ref, u_ref, c_ref, w1_ref, b1_ref, w2_ref, b2_ref,
              o_ref):
  usel = jnp.where(u_ref[...] == 1, xu_ref[:, _D:], xu_ref[:, :_D])
  csel = jnp.where(c_ref[...] == 1, xc_ref[:, _D:], xc_ref[:, :_D])
  x = jnp.concatenate([usel, csel], axis=1)
  h = lax.dot_general(x, w1_ref[...], (((1,), (1,)), ((), ())),
                      preferred_element_type=jnp.float32)
  h = jnp.maximum(h + b1_ref[...], 0.0)
  o_ref[...] = jnp.sum(h * w2_ref[...], axis=1, keepdims=True) + b2_ref[...]


def _tc_mlp(xu, xc, u_ids, c_ids, w1, b1, w2, b2):
  rows = 2048
  grid = _B // rows
  return pl.pallas_call(
      _mlp_body,
      grid=(grid,),
      in_specs=[
          pl.BlockSpec((rows, 2 * _D), lambda i: (i, 0)),
          pl.BlockSpec((rows, 2 * _D), lambda i: (i, 0)),
          pl.BlockSpec((rows, 1), lambda i: (i, 0)),
          pl.BlockSpec((rows, 1), lambda i: (i, 0)),
          pl.BlockSpec((_H, 2 * _D), lambda i: (0, 0)),
          pl.BlockSpec((1, _H), lambda i: (0, 0)),
          pl.BlockSpec((1, _H), lambda i: (0, 0)),
          pl.BlockSpec((1, 1), lambda i: (0, 0)),
      ],
      out_specs=pl.BlockSpec((rows, 1), lambda i: (i, 0)),
      out_shape=jax.ShapeDtypeStruct((_B, 1), jnp.float32),
  )(xu, xc, u_ids, c_ids, w1, b1, w2, b2)


def kernel(users, courses, user_table, course_table, W1, b1, W2, b2):
  users = users.astype(jnp.int32)
  courses = courses.astype(jnp.int32)
  uidx2d = ((users // _PBLK) * _PHALF + (users % _PHALF)).reshape(
      _NW * _NCHUNK, _CHUNK)
  cidx2d = ((courses // _PBLK) * _PHALF + (courses % _PHALF)).reshape(
      _NW * _NCHUNK, _CHUNK)
  uhalf = ((users // _PHALF) & 1).reshape(_B, 1)
  chalf = ((courses // _PHALF) & 1).reshape(_B, 1)
  ct2 = _tc_pack(course_table.T)
  xc = _sc_gather(cidx2d, ct2)
  ut2 = _tc_pack(user_table.T)
  xu = _sc_gather(uidx2d, ut2)
  return _tc_mlp(xu, xc, uhalf, chalf,
                 W1, b1.reshape(1, _H), W2, b2.reshape(1, 1))


# speedup vs baseline: 3.4146x; 1.2331x over previous
import functools

import jax
import jax.numpy as jnp
from jax import lax
from jax.experimental import pallas as pl
from jax.experimental.pallas import tpu as pltpu
from jax.experimental.pallas import tpu_sc as plsc

_NC = 2
_NS = 16
_NW = _NC * _NS

_B = 16384
_D = 64
_H = 128
_BPW = _B // _NW
_CHUNK = 128
_NCHUNK = _BPW // _CHUNK


_PBLK = 32768
_PHALF = _PBLK // 2


_PQ = _PHALF // 2


def _pack_body(x_ref, o_ref):
  x = x_ref[...]
  x2 = jnp.concatenate([x[:, :_PHALF], x[:, _PHALF:]], axis=0)
  t_a = jnp.transpose(x2[:, :_PQ])
  eye = jnp.eye(2 * _D, dtype=jnp.float32)
  t_b = lax.dot_general(x2[:, _PQ:], eye, (((0,), (0,)), ((), ())),
                        preferred_element_type=jnp.float32)
  o_ref[pl.ds(0, _PQ), :] = t_a
  o_ref[pl.ds(_PQ, _PHALF - _PQ), :] = t_b


def _tc_pack(table_t):
  n = table_t.shape[1]
  grid = (n + _PBLK - 1) // _PBLK
  return pl.pallas_call(
      _pack_body,
      grid=(grid,),
      in_specs=[pl.BlockSpec((_D, _PBLK), lambda i: (0, i))],
      out_specs=pl.BlockSpec((_PHALF, 2 * _D), lambda i: (i, 0)),
      out_shape=jax.ShapeDtypeStruct((grid * _PHALF, 2 * _D), jnp.float32),
  )(table_t)


def _sc_gather(idx2d, tbl):
  mesh = plsc.VectorSubcoreMesh(core_axis_name="c", subcore_axis_name="s")

  @functools.partial(
      pl.kernel,
      out_type=jax.ShapeDtypeStruct((_B, 2 * _D), jnp.float32),
      mesh=mesh,
      scratch_types=[
          pltpu.VMEM((_NCHUNK, _CHUNK), jnp.int32),
          pltpu.VMEM((2 * _CHUNK, 2 * _D), jnp.float32),
          pltpu.SemaphoreType.DMA,
      ],
  )
  def k(idx_hbm, tbl_hbm, out, idx_v, win_v, sem):
    wid = lax.axis_index("s") * _NC + lax.axis_index("c")
    base = wid * _BPW
    pltpu.sync_copy(idx_hbm.at[pl.ds(wid * _NCHUNK, _NCHUNK)], idx_v)
    for h in range(2):
      copies = []
      for ch in range(2):
        copies.append(pltpu.async_copy(
            tbl_hbm.at[idx_v.at[2 * h + ch]],
            win_v.at[pl.ds(ch * _CHUNK, _CHUNK)], sem))
      for cp in copies:
        cp.wait()
      pltpu.sync_copy(win_v, out.at[pl.ds(base + h * 2 * _CHUNK,
                                          2 * _CHUNK)])

  return k(idx2d, tbl)


def _mlp_body(xu_ref, xc_ref, u_ref, c_ref, w1_ref, b1_ref, w2_ref, b2_ref,
              o_ref):
  usel = jnp.where(u_ref[...] == 1, xu_ref[:, _D:], xu_ref[:, :_D])
  csel = jnp.where(c_ref[...] == 1, xc_ref[:, _D:], xc_ref[:, :_D])
  x = jnp.concatenate([usel, csel], axis=1)
  h = lax.dot_general(x, w1_ref[...], (((1,), (1,)), ((), ())),
                      preferred_element_type=jnp.float32)
  h = jnp.maximum(h + b1_ref[...], 0.0)
  o_ref[...] = jnp.sum(h * w2_ref[...], axis=1, keepdims=True) + b2_ref[...]


def _tc_mlp(xu, xc, u_ids, c_ids, w1, b1, w2, b2):
  rows = 2048
  grid = _B // rows
  return pl.pallas_call(
      _mlp_body,
      grid=(grid,),
      in_specs=[
          pl.BlockSpec((rows, 2 * _D), lambda i: (i, 0)),
          pl.BlockSpec((rows, 2 * _D), lambda i: (i, 0)),
          pl.BlockSpec((rows, 1), lambda i: (i, 0)),
          pl.BlockSpec((rows, 1), lambda i: (i, 0)),
          pl.BlockSpec((_H, 2 * _D), lambda i: (0, 0)),
          pl.BlockSpec((1, _H), lambda i: (0, 0)),
          pl.BlockSpec((1, _H), lambda i: (0, 0)),
          pl.BlockSpec((1, 1), lambda i: (0, 0)),
      ],
      out_specs=pl.BlockSpec((rows, 1), lambda i: (i, 0)),
      out_shape=jax.ShapeDtypeStruct((_B, 1), jnp.float32),
  )(xu, xc, u_ids, c_ids, w1, b1, w2, b2)


def kernel(users, courses, user_table, course_table, W1, b1, W2, b2):
  users = users.astype(jnp.int32)
  courses = courses.astype(jnp.int32)
  uidx2d = ((users // _PBLK) * _PHALF + (users % _PHALF)).reshape(
      _NW * _NCHUNK, _CHUNK)
  cidx2d = ((courses // _PBLK) * _PHALF + (courses % _PHALF)).reshape(
      _NW * _NCHUNK, _CHUNK)
  uhalf = ((users // _PHALF) & 1).reshape(_B, 1)
  chalf = ((courses // _PHALF) & 1).reshape(_B, 1)
  ct2 = _tc_pack(course_table.T)
  xc = _sc_gather(cidx2d, ct2)
  ut2 = _tc_pack(user_table.T)
  xu = _sc_gather(uidx2d, ut2)
  return _tc_mlp(xu, xc, uhalf, chalf,
                 W1, b1.reshape(1, _H), W2, b2.reshape(1, 1))


# speedup vs baseline: 3.4159x; 1.0004x over previous
import functools

import jax
import jax.numpy as jnp
from jax import lax
from jax.experimental import pallas as pl
from jax.experimental.pallas import tpu as pltpu
from jax.experimental.pallas import tpu_sc as plsc

_NC = 2
_NS = 16
_NW = _NC * _NS

_B = 16384
_D = 64
_H = 128
_BPW = _B // _NW
_CHUNK = 128
_NCHUNK = _BPW // _CHUNK


_PBLK = 32768
_PHALF = _PBLK // 2


_PQ = _PHALF // 2


def _pack_body(x_ref, o_ref):
  x = x_ref[...]
  x2 = jnp.concatenate([x[:, :_PHALF], x[:, _PHALF:]], axis=0)
  t_a = jnp.transpose(x2[:, :_PQ])
  eye = jnp.eye(2 * _D, dtype=jnp.float32)
  t_b = lax.dot_general(x2[:, _PQ:], eye, (((0,), (0,)), ((), ())),
                        preferred_element_type=jnp.float32)
  o_ref[pl.ds(0, _PQ), :] = t_a
  o_ref[pl.ds(_PQ, _PHALF - _PQ), :] = t_b


def _tc_pack(table_t):
  n = table_t.shape[1]
  grid = (n + _PBLK - 1) // _PBLK
  return pl.pallas_call(
      _pack_body,
      grid=(grid,),
      in_specs=[pl.BlockSpec((_D, _PBLK), lambda i: (0, i))],
      out_specs=pl.BlockSpec((_PHALF, 2 * _D), lambda i: (i, 0)),
      out_shape=jax.ShapeDtypeStruct((grid * _PHALF, 2 * _D), jnp.float32),
  )(table_t)


def _sc_gather(idx2d, tbl):
  mesh = plsc.VectorSubcoreMesh(core_axis_name="c", subcore_axis_name="s")

  @functools.partial(
      pl.kernel,
      out_type=jax.ShapeDtypeStruct((_B, 2 * _D), jnp.float32),
      mesh=mesh,
      scratch_types=[
          pltpu.VMEM((_NCHUNK, _CHUNK), jnp.int32),
          pltpu.VMEM((2 * _CHUNK, 2 * _D), jnp.float32),
          pltpu.SemaphoreType.DMA,
      ],
  )
  def k(idx_hbm, tbl_hbm, out, idx_v, win_v, sem):
    wid = lax.axis_index("s") * _NC + lax.axis_index("c")
    base = wid * _BPW
    pltpu.sync_copy(idx_hbm.at[pl.ds(wid * _NCHUNK, _NCHUNK)], idx_v)
    for h in range(2):
      copies = []
      for ch in range(2):
        copies.append(pltpu.async_copy(
            tbl_hbm.at[idx_v.at[2 * h + ch]],
            win_v.at[pl.ds(ch * _CHUNK, _CHUNK)], sem))
      for cp in copies:
        cp.wait()
      pltpu.sync_copy(win_v, out.at[pl.ds(base + h * 2 * _CHUNK,
                                          2 * _CHUNK)])

  return k(idx2d, tbl)


def _mlp_body(xu_ref, xc_ref, u_ref, c_ref, w1_ref, b1_ref, w2_ref, b2_ref,
              o_ref):
  usel = jnp.where(u_ref[...] == 1, xu_ref[:, _D:], xu_ref[:, :_D])
  csel = jnp.where(c_ref[...] == 1, xc_ref[:, _D:], xc_ref[:, :_D])
  x = jnp.concatenate([usel, csel], axis=1)
  h = lax.dot_general(x, w1_ref[...], (((1,), (1,)), ((), ())),
                      preferred_element_type=jnp.float32)
  h = jnp.maximum(h + b1_ref[...], 0.0)
  o_ref[...] = jnp.sum(h * w2_ref[...], axis=1, keepdims=True) + b2_ref[...]


def _tc_mlp(xu, xc, u_ids, c_ids, w1, b1, w2, b2):
  rows = 2048
  grid = _B // rows
  return pl.pallas_call(
      _mlp_body,
      grid=(grid,),
      in_specs=[
          pl.BlockSpec((rows, 2 * _D), lambda i: (i, 0)),
          pl.BlockSpec((rows, 2 * _D), lambda i: (i, 0)),
          pl.BlockSpec((rows, 1), lambda i: (i, 0)),
          pl.BlockSpec((rows, 1), lambda i: (i, 0)),
          pl.BlockSpec((_H, 2 * _D), lambda i: (0, 0)),
          pl.BlockSpec((1, _H), lambda i: (0, 0)),
          pl.BlockSpec((1, _H), lambda i: (0, 0)),
          pl.BlockSpec((1, 1), lambda i: (0, 0)),
      ],
      out_specs=pl.BlockSpec((rows, 1), lambda i: (i, 0)),
      out_shape=jax.ShapeDtypeStruct((_B, 1), jnp.float32),
  )(xu, xc, u_ids, c_ids, w1, b1, w2, b2)


def kernel(users, courses, user_table, course_table, W1, b1, W2, b2):
  users = users.astype(jnp.int32)
  courses = courses.astype(jnp.int32)
  uidx2d = ((users // _PBLK) * _PHALF + (users % _PHALF)).reshape(
      _NW * _NCHUNK, _CHUNK)
  cidx2d = ((courses // _PBLK) * _PHALF + (courses % _PHALF)).reshape(
      _NW * _NCHUNK, _CHUNK)
  uhalf = ((users // _PHALF) & 1).reshape(_B, 1)
  chalf = ((courses // _PHALF) & 1).reshape(_B, 1)
  ct2 = _tc_pack(course_table.T)
  xc = _sc_gather(cidx2d, ct2)
  user_table_t, ct2 = lax.optimization_barrier((user_table.T, ct2))
  ut2 = _tc_pack(user_table_t)
  xu = _sc_gather(uidx2d, ut2)
  return _tc_mlp(xu, xc, uhalf, chalf,
                 W1, b1.reshape(1, _H), W2, b2.reshape(1, 1))
